# baseline (speedup 1.0000x reference)
import jax
import jax.numpy as jnp
from jax import lax
from jax.experimental import pallas as pl
from jax.experimental.pallas import tpu as pltpu
from jax.experimental.pallas import tpu_sc as plsc

N = 10000
NP = 10240
E = 320000
NG = 64
D_IN = 128
DH = 256
D_OUT = 64
HALF = 128
WIN = 184
EPT = 20056
EP = 16 * EPT
DWIN = 2000
ZCH = NP // 16
ZR = 128
TR = 1024
GRID = NP // TR

_HIGH = jax.lax.Precision.HIGHEST


def _mesh():
    return plsc.VectorSubcoreMesh(core_axis_name="c", subcore_axis_name="s")


def _dot(a, b):
    return lax.dot_general(a, b, (((1,), (0,)), ((), ())),
                           preferred_element_type=jnp.float32)


def _sc_deg_body(dst_hbm, out0_hbm, out1_hbm, dstv, onesv, zerov, deg_sh):
    c = lax.axis_index("c")
    s = lax.axis_index("s")
    for k in range(DWIN // 16):
        onesv[pl.ds(16 * k, 16)] = jnp.full((16,), 1.0, jnp.float32)
    for k in range(ZCH // 16):
        zerov[pl.ds(16 * k, 16)] = jnp.zeros((16,), jnp.float32)
    pltpu.sync_copy(zerov, deg_sh.at[pl.ds(s * ZCH, ZCH)])
    plsc.subcore_barrier()
    epw = E // 32
    base = (c * 16 + s) * epw

    def body(w, carry):
        pltpu.sync_copy(dst_hbm.at[pl.ds(base + w * DWIN, DWIN)], dstv)
        pltpu.sync_copy(onesv, deg_sh.at[dstv], add=True)
        return carry

    lax.fori_loop(0, epw // DWIN, body, 0)
    plsc.subcore_barrier()

    @pl.when(c == 0)
    def _():
        pltpu.sync_copy(deg_sh.at[pl.ds(s * ZCH, ZCH)],
                        out0_hbm.at[pl.ds(s * ZCH, ZCH)])

    @pl.when(c == 1)
    def _():
        pltpu.sync_copy(deg_sh.at[pl.ds(s * ZCH, ZCH)],
                        out1_hbm.at[pl.ds(s * ZCH, ZCH)])


def _sc_deg():
    return pl.kernel(
        _sc_deg_body, mesh=_mesh(),
        out_type=[jax.ShapeDtypeStruct((NP,), jnp.float32),
                  jax.ShapeDtypeStruct((NP,), jnp.float32)],
        scratch_types=[pltpu.VMEM((DWIN,), jnp.int32),
                       pltpu.VMEM((DWIN,), jnp.float32),
                       pltpu.VMEM((ZCH,), jnp.float32),
                       pltpu.VMEM_SHARED((NP,), jnp.float32)],
    )


NB = 2


def _sc_agg_body(g_hbm, src_hbm, dst_hbm, out_hbm, srcs, dsts, msgs,
                 sem_si, sem_sd, sem_g, sem_s, acc_sh):
    c = lax.axis_index("c")
    s = lax.axis_index("s")
    rb = s * ZCH
    ebase = s * EPT
    nw = EPT // WIN
    gq = g_hbm.at[c]
    pltpu.sync_copy(g_hbm.at[c, pl.ds(rb, ZCH)], acc_sh.at[pl.ds(rb, ZCH)])
    plsc.subcore_barrier()

    def src_at(w):
        return src_hbm.at[pl.ds(ebase + w * WIN, WIN)]

    def dst_at(w):
        return dst_hbm.at[pl.ds(ebase + w * WIN, WIN)]

    pltpu.async_copy(src_at(0), srcs.at[0], sem_si.at[0]).wait()
    pltpu.async_copy(gq.at[srcs.at[0]], msgs.at[0], sem_g.at[0])
    pltpu.async_copy(src_at(1), srcs.at[1], sem_si.at[1])
    pltpu.async_copy(dst_at(0), dsts.at[0], sem_sd.at[0])

    def win(w, first, has_next, b):
        pltpu.make_async_copy(gq.at[srcs.at[b]], msgs.at[b],
                              sem_g.at[b]).wait()
        @pl.when(jnp.logical_not(first))
        def _():
            pltpu.make_async_copy(msgs.at[1 - b], acc_sh.at[dsts.at[1 - b]],
                                  sem_s.at[1 - b]).wait()
        if has_next:
            pltpu.make_async_copy(src_at(w + 1), srcs.at[1 - b],
                                  sem_si.at[1 - b]).wait()
            pltpu.async_copy(gq.at[srcs.at[1 - b]], msgs.at[1 - b],
                             sem_g.at[1 - b])
            @pl.when(jnp.asarray(w + 2 < nw))
            def _():
                pltpu.async_copy(src_at(w + 2), srcs.at[b], sem_si.at[b])
            pltpu.async_copy(dst_at(w + 1), dsts.at[1 - b], sem_sd.at[1 - b])
        pltpu.make_async_copy(dst_at(w), dsts.at[b], sem_sd.at[b]).wait()
        pltpu.async_copy(msgs.at[b], acc_sh.at[dsts.at[b]], sem_s.at[b],
                         add=True)

    def body(i, carry):
        wo = i * 2
        win(wo, wo == 0, True, 0)
        win(wo + 1, jnp.bool_(False), True, 1)
        return carry

    lax.fori_loop(0, (nw - 1) // 2, body, 0)
    win(nw - 1, jnp.bool_(False), False, 0)
    pltpu.make_async_copy(msgs.at[0], acc_sh.at[dsts.at[0]],
                          sem_s.at[0]).wait()
    plsc.subcore_barrier()
    pltpu.sync_copy(acc_sh.at[pl.ds(rb, ZCH)],
                    out_hbm.at[pl.ds(c * NP + rb, ZCH)])


def _sc_agg():
    return pl.kernel(
        _sc_agg_body, mesh=_mesh(),
        compiler_params=pltpu.CompilerParams(use_tc_tiling_on_sc=False),
        out_type=jax.ShapeDtypeStruct((2 * NP, HALF), jnp.float32),
        scratch_types=[pltpu.VMEM((NB, WIN), jnp.int32),
                       pltpu.VMEM((NB, WIN), jnp.int32),
                       pltpu.VMEM((NB, WIN, HALF), jnp.float32),
                       pltpu.SemaphoreType.DMA((NB,)),
                       pltpu.SemaphoreType.DMA((NB,)),
                       pltpu.SemaphoreType.DMA((NB,)),
                       pltpu.SemaphoreType.DMA((NB,)),
                       pltpu.VMEM_SHARED((NP, HALF), jnp.float32)],
    )


def _split2(g, g_ref):
    for j in range(2):
        g_ref[j] = g[:, j * HALF:(j + 1) * HALF]


def _first_body(x_ref, w_ref, d0_ref, d1_ref, g_ref, dinv_ref):
    dinv = lax.rsqrt(d0_ref[...] + d1_ref[...] + 1.0)
    dinv_ref[...] = dinv
    g = dinv * _dot(x_ref[...], w_ref[...])
    _split2(g, g_ref)


def _mid_body(s_ref, dinv_ref, b_ref, w_ref, g_ref):
    dinv = dinv_ref[...]
    h = jnp.concatenate([s_ref[j] for j in range(2)], axis=1)
    h = jnp.maximum(dinv * h + b_ref[...], 0.0)
    g = dinv * _dot(h, w_ref[...])
    _split2(g, g_ref)


def _pool_body(s_ref, dinv_ref, b_ref, batch_ref, fw1_ref, fb1_ref,
               fw2_ref, fb2_ref, out_ref, sums, counts):
    i = pl.program_id(0)

    @pl.when(i == 0)
    def _():
        sums[...] = jnp.zeros_like(sums)
        counts[...] = jnp.zeros_like(counts)

    h = jnp.concatenate([s_ref[j] for j in range(2)], axis=1)
    h = jnp.maximum(dinv_ref[...] * h + b_ref[...], 0.0)
    gid = lax.broadcasted_iota(jnp.int32, (TR, NG), 1)
    oh = (batch_ref[...] == gid).astype(jnp.float32)
    sums[...] += lax.dot_general(oh, h, (((0,), (0,)), ((), ())),
                                 preferred_element_type=jnp.float32)
    counts[...] += lax.dot_general(oh, jnp.ones((TR, 1), jnp.float32),
                                   (((0,), (0,)), ((), ())),
                                   preferred_element_type=jnp.float32)

    @pl.when(i == pl.num_programs(0) - 1)
    def _():
        pooled = sums[...] / jnp.maximum(counts[...], 1.0)
        h2 = jnp.maximum(_dot(pooled, fw1_ref[...]) + fb1_ref[...], 0.0)
        out_ref[...] = _dot(h2, fw2_ref[...]) + fb2_ref[...]


def _row_spec(width):
    return pl.BlockSpec((TR, width), lambda i: (i, 0))


_SPLIT_SPEC = pl.BlockSpec((2, TR, HALF), lambda i: (0, i, 0))
_FULL = lambda shape: pl.BlockSpec(shape, lambda i: tuple(0 for _ in shape))


def _tc_first(xp, W1, deg0, deg1):
    return pl.pallas_call(
        _first_body,
        grid=(GRID,),
        in_specs=[_row_spec(D_IN), _FULL((D_IN, DH)), _row_spec(1), _row_spec(1)],
        out_specs=[_SPLIT_SPEC, _row_spec(1)],
        out_shape=[jax.ShapeDtypeStruct((2, NP, HALF), jnp.float32),
                   jax.ShapeDtypeStruct((NP, 1), jnp.float32)],
    )(xp, W1, deg0, deg1)


def _tc_mid(s, dinv, b, W):
    return pl.pallas_call(
        _mid_body,
        grid=(GRID,),
        in_specs=[_SPLIT_SPEC, _row_spec(1), _FULL((1, DH)), _FULL((DH, DH))],
        out_specs=_SPLIT_SPEC,
        out_shape=jax.ShapeDtypeStruct((2, NP, HALF), jnp.float32),
    )(s, dinv, b, W)


def _tc_pool(s, dinv, b, batch_p, fw1, fb1, fw2, fb2):
    return pl.pallas_call(
        _pool_body,
        grid=(GRID,),
        in_specs=[_SPLIT_SPEC, _row_spec(1), _FULL((1, DH)),
                  _row_spec(1), _FULL((DH, DH)), _FULL((1, DH)),
                  _FULL((DH, D_OUT)), _FULL((1, D_OUT))],
        out_specs=pl.BlockSpec((NG, D_OUT), lambda i: (0, 0)),
        out_shape=jax.ShapeDtypeStruct((NG, D_OUT), jnp.float32),
        scratch_shapes=[pltpu.VMEM((NG, DH), jnp.float32),
                        pltpu.VMEM((NG, 1), jnp.float32)],
    )(s, dinv, b, batch_p, fw1, fb1, fw2, fb2)


def kernel(x, edge_index, batch, W1, b1, W2, b2, W3, b3, W4, b4, W5, b5,
           fw1, fb1, fw2, fb2):
    src = edge_index[0]
    dst = edge_index[1]
    srcp = jnp.pad(src, (0, EP - E))
    dstp = jnp.pad(dst, (0, EP - E), constant_values=NP - 1)
    xp = jnp.pad(x, ((0, NP - N), (0, 0)))
    batch_p = jnp.pad(batch, (0, NP - N), constant_values=NG).reshape(NP, 1)

    deg0, deg1 = _sc_deg()(dst)
    g, dinv = _tc_first(xp, W1, deg0.reshape(NP, 1), deg1.reshape(NP, 1))
    for Wn, bp in ((W2, b1), (W3, b2), (W4, b3), (W5, b4)):
        s = _sc_agg()(g, srcp, dstp).reshape(2, NP, HALF)
        g = _tc_mid(s, dinv, bp.reshape(1, DH), Wn)
    s = _sc_agg()(g, srcp, dstp).reshape(2, NP, HALF)
    return _tc_pool(s, dinv, b5.reshape(1, DH), batch_p, fw1,
                    fb1.reshape(1, DH), fw2, fb2.reshape(1, D_OUT))


# speedup vs baseline: 1.0147x; 1.0147x over previous
import jax
import jax.numpy as jnp
from jax import lax
from jax.experimental import pallas as pl
from jax.experimental.pallas import tpu as pltpu
from jax.experimental.pallas import tpu_sc as plsc

N = 10000
NP = 10240
E = 320000
NG = 64
D_IN = 128
DH = 256
D_OUT = 64
HALF = 128
WIN = 184
EPT = 20056
EP = 16 * EPT
DWIN = 2000
ZCH = NP // 16
ZR = 128
TR = 1024
GRID = NP // TR

_HIGH = jax.lax.Precision.HIGHEST


def _mesh():
    return plsc.VectorSubcoreMesh(core_axis_name="c", subcore_axis_name="s")


def _dot(a, b):
    return lax.dot_general(a, b, (((1,), (0,)), ((), ())),
                           preferred_element_type=jnp.float32)


def _sc_deg_body(dst_hbm, out0_hbm, out1_hbm, dstv, onesv, zerov, deg_sh):
    c = lax.axis_index("c")
    s = lax.axis_index("s")
    for k in range(DWIN // 16):
        onesv[pl.ds(16 * k, 16)] = jnp.full((16,), 1.0, jnp.float32)
    for k in range(ZCH // 16):
        zerov[pl.ds(16 * k, 16)] = jnp.zeros((16,), jnp.float32)
    pltpu.sync_copy(zerov, deg_sh.at[pl.ds(s * ZCH, ZCH)])
    plsc.subcore_barrier()
    epw = E // 32
    base = (c * 16 + s) * epw

    def body(w, carry):
        pltpu.sync_copy(dst_hbm.at[pl.ds(base + w * DWIN, DWIN)], dstv)
        pltpu.sync_copy(onesv, deg_sh.at[dstv], add=True)
        return carry

    lax.fori_loop(0, epw // DWIN, body, 0)
    plsc.subcore_barrier()

    @pl.when(c == 0)
    def _():
        pltpu.sync_copy(deg_sh.at[pl.ds(s * ZCH, ZCH)],
                        out0_hbm.at[pl.ds(s * ZCH, ZCH)])

    @pl.when(c == 1)
    def _():
        pltpu.sync_copy(deg_sh.at[pl.ds(s * ZCH, ZCH)],
                        out1_hbm.at[pl.ds(s * ZCH, ZCH)])


def _sc_deg():
    return pl.kernel(
        _sc_deg_body, mesh=_mesh(),
        out_type=[jax.ShapeDtypeStruct((NP,), jnp.float32),
                  jax.ShapeDtypeStruct((NP,), jnp.float32)],
        scratch_types=[pltpu.VMEM((DWIN,), jnp.int32),
                       pltpu.VMEM((DWIN,), jnp.float32),
                       pltpu.VMEM((ZCH,), jnp.float32),
                       pltpu.VMEM_SHARED((NP,), jnp.float32)],
    )


NB = 2


def _sc_agg_body(g_hbm, src_hbm, dst_hbm, out_hbm, srcs, dsts, msgs,
                 sem_si, sem_sd, sem_g, sem_s, acc_sh):
    c = lax.axis_index("c")
    s = lax.axis_index("s")
    rb = s * ZCH
    ebase = s * EPT
    nw = EPT // WIN
    gq = g_hbm.at[c]
    pltpu.sync_copy(g_hbm.at[c, pl.ds(rb, ZCH)], acc_sh.at[pl.ds(rb, ZCH)])
    plsc.subcore_barrier()

    def src_at(w):
        return src_hbm.at[pl.ds(ebase + w * WIN, WIN)]

    def dst_at(w):
        return dst_hbm.at[pl.ds(ebase + w * WIN, WIN)]

    pltpu.async_copy(src_at(0), srcs.at[0], sem_si.at[0]).wait()
    pltpu.async_copy(gq.at[srcs.at[0]], msgs.at[0], sem_g.at[0])
    pltpu.async_copy(src_at(1), srcs.at[1], sem_si.at[1])
    pltpu.async_copy(dst_at(0), dsts.at[0], sem_sd.at[0])

    def win(w, first, has_next, b):
        pltpu.make_async_copy(gq.at[srcs.at[b]], msgs.at[b],
                              sem_g.at[b]).wait()
        @pl.when(jnp.logical_not(first))
        def _():
            pltpu.make_async_copy(msgs.at[1 - b], acc_sh.at[dsts.at[1 - b]],
                                  sem_s.at[1 - b]).wait()
        if has_next:
            pltpu.make_async_copy(src_at(w + 1), srcs.at[1 - b],
                                  sem_si.at[1 - b]).wait()
            pltpu.async_copy(gq.at[srcs.at[1 - b]], msgs.at[1 - b],
                             sem_g.at[1 - b])
            @pl.when(jnp.asarray(w + 2 < nw))
            def _():
                pltpu.async_copy(src_at(w + 2), srcs.at[b], sem_si.at[b])
            pltpu.async_copy(dst_at(w + 1), dsts.at[1 - b], sem_sd.at[1 - b])
        pltpu.make_async_copy(dst_at(w), dsts.at[b], sem_sd.at[b]).wait()
        pltpu.async_copy(msgs.at[b], acc_sh.at[dsts.at[b]], sem_s.at[b],
                         add=True)

    def body(i, carry):
        wo = i * 2
        win(wo, wo == 0, True, 0)
        win(wo + 1, jnp.bool_(False), True, 1)
        return carry

    lax.fori_loop(0, (nw - 1) // 2, body, 0)
    win(nw - 1, jnp.bool_(False), False, 0)
    pltpu.make_async_copy(msgs.at[0], acc_sh.at[dsts.at[0]],
                          sem_s.at[0]).wait()
    plsc.subcore_barrier()
    pltpu.sync_copy(acc_sh.at[pl.ds(rb, ZCH)],
                    out_hbm.at[pl.ds(c * NP + rb, ZCH)])


def _sc_agg():
    return pl.kernel(
        _sc_agg_body, mesh=_mesh(),
        compiler_params=pltpu.CompilerParams(use_tc_tiling_on_sc=False),
        out_type=jax.ShapeDtypeStruct((2 * NP, HALF), jnp.float32),
        scratch_types=[pltpu.VMEM((NB, WIN), jnp.int32),
                       pltpu.VMEM((NB, WIN), jnp.int32),
                       pltpu.VMEM((NB, WIN, HALF), jnp.float32),
                       pltpu.SemaphoreType.DMA((NB,)),
                       pltpu.SemaphoreType.DMA((NB,)),
                       pltpu.SemaphoreType.DMA((NB,)),
                       pltpu.SemaphoreType.DMA((NB,)),
                       pltpu.VMEM_SHARED((NP, HALF), jnp.float32)],
    )


def _split2(g, g_ref):
    for j in range(2):
        g_ref[j] = g[:, j * HALF:(j + 1) * HALF]


def _first_body(x_ref, w_ref, d0_ref, d1_ref, g_ref, dinv_ref):
    dinv = lax.rsqrt(d0_ref[...] + d1_ref[...] + 1.0)
    dinv_ref[...] = dinv
    g = dinv * _dot(x_ref[...], w_ref[...])
    _split2(g, g_ref)


def _mid_body(s_ref, dinv_ref, b_ref, w_ref, g_ref):
    dinv = dinv_ref[...]
    h = jnp.concatenate([s_ref[j] for j in range(2)], axis=1)
    h = jnp.maximum(dinv * h + b_ref[...], 0.0)
    g = dinv * _dot(h, w_ref[...])
    _split2(g, g_ref)


def _pool_body(s_ref, dinv_ref, b_ref, batch_ref, fw1_ref, fb1_ref,
               fw2_ref, fb2_ref, out_ref, sums, counts):
    i = pl.program_id(0)

    @pl.when(i == 0)
    def _():
        sums[...] = jnp.zeros_like(sums)
        counts[...] = jnp.zeros_like(counts)

    h = jnp.concatenate([s_ref[j] for j in range(2)], axis=1)
    h = jnp.maximum(dinv_ref[...] * h + b_ref[...], 0.0)
    gid = lax.broadcasted_iota(jnp.int32, (TR, NG), 1)
    oh = (batch_ref[...] == gid).astype(jnp.float32)
    sums[...] += lax.dot_general(oh, h, (((0,), (0,)), ((), ())),
                                 preferred_element_type=jnp.float32)
    counts[...] += lax.dot_general(oh, jnp.ones((TR, 1), jnp.float32),
                                   (((0,), (0,)), ((), ())),
                                   preferred_element_type=jnp.float32)

    @pl.when(i == pl.num_programs(0) - 1)
    def _():
        pooled = sums[...] / jnp.maximum(counts[...], 1.0)
        h2 = jnp.maximum(_dot(pooled, fw1_ref[...]) + fb1_ref[...], 0.0)
        out_ref[...] = _dot(h2, fw2_ref[...]) + fb2_ref[...]


def _row_spec(width):
    return pl.BlockSpec((TR, width), lambda i: (i, 0))


_SPLIT_SPEC = pl.BlockSpec((2, TR, HALF), lambda i: (0, i, 0))
_FULL = lambda shape: pl.BlockSpec(shape, lambda i: tuple(0 for _ in shape))


def _tc_first(xp, W1, deg0, deg1):
    return pl.pallas_call(
        _first_body,
        grid=(GRID,),
        in_specs=[_row_spec(D_IN), _FULL((D_IN, DH)), _row_spec(1), _row_spec(1)],
        out_specs=[_SPLIT_SPEC, _row_spec(1)],
        out_shape=[jax.ShapeDtypeStruct((2, NP, HALF), jnp.float32),
                   jax.ShapeDtypeStruct((NP, 1), jnp.float32)],
    )(xp, W1, deg0, deg1)


def _tc_mid(s, dinv, b, W):
    return pl.pallas_call(
        _mid_body,
        grid=(GRID,),
        in_specs=[_SPLIT_SPEC, _row_spec(1), _FULL((1, DH)), _FULL((DH, DH))],
        out_specs=_SPLIT_SPEC,
        out_shape=jax.ShapeDtypeStruct((2, NP, HALF), jnp.float32),
    )(s, dinv, b, W)


def _tc_pool(s, dinv, b, batch_p, fw1, fb1, fw2, fb2):
    return pl.pallas_call(
        _pool_body,
        grid=(GRID,),
        in_specs=[_SPLIT_SPEC, _row_spec(1), _FULL((1, DH)),
                  _row_spec(1), _FULL((DH, DH)), _FULL((1, DH)),
                  _FULL((DH, D_OUT)), _FULL((1, D_OUT))],
        out_specs=pl.BlockSpec((NG, D_OUT), lambda i: (0, 0)),
        out_shape=jax.ShapeDtypeStruct((NG, D_OUT), jnp.float32),
        scratch_shapes=[pltpu.VMEM((NG, DH), jnp.float32),
                        pltpu.VMEM((NG, 1), jnp.float32)],
    )(s, dinv, b, batch_p, fw1, fb1, fw2, fb2)


def kernel(x, edge_index, batch, W1, b1, W2, b2, W3, b3, W4, b4, W5, b5,
           fw1, fb1, fw2, fb2):
    src = edge_index[0]
    dst = edge_index[1]
    srcp = jnp.pad(src, (0, EP - E))
    pad_rows = N + (jnp.arange(EP - E, dtype=jnp.int32) % (NP - N))
    dstp = jnp.concatenate([dst, pad_rows])
    xp = jnp.pad(x, ((0, NP - N), (0, 0)))
    batch_p = jnp.pad(batch, (0, NP - N), constant_values=NG).reshape(NP, 1)

    deg0, deg1 = _sc_deg()(dst)
    g, dinv = _tc_first(xp, W1, deg0.reshape(NP, 1), deg1.reshape(NP, 1))
    for Wn, bp in ((W2, b1), (W3, b2), (W4, b3), (W5, b4)):
        s = _sc_agg()(g, srcp, dstp).reshape(2, NP, HALF)
        g = _tc_mid(s, dinv, bp.reshape(1, DH), Wn)
    s = _sc_agg()(g, srcp, dstp).reshape(2, NP, HALF)
    return _tc_pool(s, dinv, b5.reshape(1, DH), batch_p, fw1,
                    fb1.reshape(1, DH), fw2, fb2.reshape(1, D_OUT))


# speedup vs baseline: 1.2055x; 1.1880x over previous
import jax
import jax.numpy as jnp
from jax import lax
from jax.experimental import pallas as pl
from jax.experimental.pallas import tpu as pltpu
from jax.experimental.pallas import tpu_sc as plsc

N = 10000
NP = 10240
E = 320000
NG = 64
D_IN = 128
DH = 256
D_OUT = 64
HALF = 128
WIN = 160
EPT = E // 16
DWIN = 2000
ZCH = NP // 16
ZR = 128
TR = 1024
GRID = NP // TR

_HIGH = jax.lax.Precision.HIGHEST


def _mesh():
    return plsc.VectorSubcoreMesh(core_axis_name="c", subcore_axis_name="s")


def _dot(a, b):
    return lax.dot_general(a, b, (((1,), (0,)), ((), ())),
                           preferred_element_type=jnp.float32)


def _sc_deg_body(dst_hbm, out0_hbm, out1_hbm, dstv, onesv, zerov, deg_sh):
    c = lax.axis_index("c")
    s = lax.axis_index("s")
    for k in range(DWIN // 16):
        onesv[pl.ds(16 * k, 16)] = jnp.full((16,), 1.0, jnp.float32)
    for k in range(ZCH // 16):
        zerov[pl.ds(16 * k, 16)] = jnp.zeros((16,), jnp.float32)
    pltpu.sync_copy(zerov, deg_sh.at[pl.ds(s * ZCH, ZCH)])
    plsc.subcore_barrier()
    epw = E // 32
    base = (c * 16 + s) * epw

    def body(w, carry):
        pltpu.sync_copy(dst_hbm.at[pl.ds(base + w * DWIN, DWIN)], dstv)
        pltpu.sync_copy(onesv, deg_sh.at[dstv], add=True)
        return carry

    lax.fori_loop(0, epw // DWIN, body, 0)
    plsc.subcore_barrier()

    @pl.when(c == 0)
    def _():
        pltpu.sync_copy(deg_sh.at[pl.ds(s * ZCH, ZCH)],
                        out0_hbm.at[pl.ds(s * ZCH, ZCH)])

    @pl.when(c == 1)
    def _():
        pltpu.sync_copy(deg_sh.at[pl.ds(s * ZCH, ZCH)],
                        out1_hbm.at[pl.ds(s * ZCH, ZCH)])


def _sc_deg():
    return pl.kernel(
        _sc_deg_body, mesh=_mesh(),
        out_type=[jax.ShapeDtypeStruct((NP,), jnp.float32),
                  jax.ShapeDtypeStruct((NP,), jnp.float32)],
        scratch_types=[pltpu.VMEM((DWIN,), jnp.int32),
                       pltpu.VMEM((DWIN,), jnp.float32),
                       pltpu.VMEM((ZCH,), jnp.float32),
                       pltpu.VMEM_SHARED((NP,), jnp.float32)],
    )


NB = 2


def _sc_agg_body(g_hbm, src_hbm, dst_hbm, out_hbm, srcs, dsts, msgs,
                 sem_si, sem_sd, sem_g, sem_s, acc_sh):
    c = lax.axis_index("c")
    s = lax.axis_index("s")
    rb = s * ZCH
    ebase = s * EPT
    nw = EPT // WIN
    gq = g_hbm.at[c]
    pltpu.sync_copy(g_hbm.at[c, pl.ds(rb, ZCH)], acc_sh.at[pl.ds(rb, ZCH)])
    plsc.subcore_barrier()

    def src_at(w):
        return src_hbm.at[pl.ds(ebase + w * WIN, WIN)]

    def dst_at(w):
        return dst_hbm.at[pl.ds(ebase + w * WIN, WIN)]

    pltpu.async_copy(src_at(0), srcs.at[0], sem_si.at[0]).wait()
    pltpu.async_copy(gq.at[srcs.at[0]], msgs.at[0], sem_g.at[0])
    pltpu.async_copy(src_at(1), srcs.at[1], sem_si.at[1])
    pltpu.async_copy(dst_at(0), dsts.at[0], sem_sd.at[0])

    def win(w, first, has_next, b):
        pltpu.make_async_copy(gq.at[srcs.at[b]], msgs.at[b],
                              sem_g.at[b]).wait()
        @pl.when(jnp.logical_not(first))
        def _():
            pltpu.make_async_copy(msgs.at[1 - b], acc_sh.at[dsts.at[1 - b]],
                                  sem_s.at[1 - b]).wait()
        if has_next:
            pltpu.make_async_copy(src_at(w + 1), srcs.at[1 - b],
                                  sem_si.at[1 - b]).wait()
            pltpu.async_copy(gq.at[srcs.at[1 - b]], msgs.at[1 - b],
                             sem_g.at[1 - b])
            @pl.when(jnp.asarray(w + 2 < nw))
            def _():
                pltpu.async_copy(src_at(w + 2), srcs.at[b], sem_si.at[b])
            pltpu.async_copy(dst_at(w + 1), dsts.at[1 - b], sem_sd.at[1 - b])
        pltpu.make_async_copy(dst_at(w), dsts.at[b], sem_sd.at[b]).wait()
        pltpu.async_copy(msgs.at[b], acc_sh.at[dsts.at[b]], sem_s.at[b],
                         add=True)

    def body(i, carry):
        wo = i * 2
        win(wo, wo == 0, True, 0)
        win(wo + 1, jnp.bool_(False), True, 1)
        return carry

    lax.fori_loop(0, (nw - 1) // 2, body, 0)
    win(nw - 1, jnp.bool_(False), False, 0)
    pltpu.make_async_copy(msgs.at[0], acc_sh.at[dsts.at[0]],
                          sem_s.at[0]).wait()
    plsc.subcore_barrier()
    pltpu.sync_copy(acc_sh.at[pl.ds(rb, ZCH)],
                    out_hbm.at[pl.ds(c * NP + rb, ZCH)])


def _sc_agg():
    return pl.kernel(
        _sc_agg_body, mesh=_mesh(),
        compiler_params=pltpu.CompilerParams(use_tc_tiling_on_sc=False),
        out_type=jax.ShapeDtypeStruct((2 * NP, HALF), jnp.float32),
        scratch_types=[pltpu.VMEM((NB, WIN), jnp.int32),
                       pltpu.VMEM((NB, WIN), jnp.int32),
                       pltpu.VMEM((NB, WIN, HALF), jnp.float32),
                       pltpu.SemaphoreType.DMA((NB,)),
                       pltpu.SemaphoreType.DMA((NB,)),
                       pltpu.SemaphoreType.DMA((NB,)),
                       pltpu.SemaphoreType.DMA((NB,)),
                       pltpu.VMEM_SHARED((NP, HALF), jnp.float32)],
    )


def _split2(g, g_ref):
    for j in range(2):
        g_ref[j] = g[:, j * HALF:(j + 1) * HALF]


def _first_body(x_ref, w_ref, d0_ref, d1_ref, g_ref, dinv_ref):
    dinv = lax.rsqrt(d0_ref[...] + d1_ref[...] + 1.0)
    dinv_ref[...] = dinv
    g = dinv * _dot(x_ref[...], w_ref[...])
    _split2(g, g_ref)


def _mid_body(s_ref, dinv_ref, b_ref, w_ref, g_ref):
    dinv = dinv_ref[...]
    h = jnp.concatenate([s_ref[j] for j in range(2)], axis=1)
    h = jnp.maximum(dinv * h + b_ref[...], 0.0)
    g = dinv * _dot(h, w_ref[...])
    _split2(g, g_ref)


def _pool_body(s_ref, dinv_ref, b_ref, batch_ref, fw1_ref, fb1_ref,
               fw2_ref, fb2_ref, out_ref, sums, counts):
    i = pl.program_id(0)

    @pl.when(i == 0)
    def _():
        sums[...] = jnp.zeros_like(sums)
        counts[...] = jnp.zeros_like(counts)

    h = jnp.concatenate([s_ref[j] for j in range(2)], axis=1)
    h = jnp.maximum(dinv_ref[...] * h + b_ref[...], 0.0)
    gid = lax.broadcasted_iota(jnp.int32, (TR, NG), 1)
    oh = (batch_ref[...] == gid).astype(jnp.float32)
    sums[...] += lax.dot_general(oh, h, (((0,), (0,)), ((), ())),
                                 preferred_element_type=jnp.float32)
    counts[...] += lax.dot_general(oh, jnp.ones((TR, 1), jnp.float32),
                                   (((0,), (0,)), ((), ())),
                                   preferred_element_type=jnp.float32)

    @pl.when(i == pl.num_programs(0) - 1)
    def _():
        pooled = sums[...] / jnp.maximum(counts[...], 1.0)
        h2 = jnp.maximum(_dot(pooled, fw1_ref[...]) + fb1_ref[...], 0.0)
        out_ref[...] = _dot(h2, fw2_ref[...]) + fb2_ref[...]


def _row_spec(width):
    return pl.BlockSpec((TR, width), lambda i: (i, 0))


_SPLIT_SPEC = pl.BlockSpec((2, TR, HALF), lambda i: (0, i, 0))
_FULL = lambda shape: pl.BlockSpec(shape, lambda i: tuple(0 for _ in shape))


def _tc_first(xp, W1, deg0, deg1):
    return pl.pallas_call(
        _first_body,
        grid=(GRID,),
        in_specs=[_row_spec(D_IN), _FULL((D_IN, DH)), _row_spec(1), _row_spec(1)],
        out_specs=[_SPLIT_SPEC, _row_spec(1)],
        out_shape=[jax.ShapeDtypeStruct((2, NP, HALF), jnp.float32),
                   jax.ShapeDtypeStruct((NP, 1), jnp.float32)],
    )(xp, W1, deg0, deg1)


def _tc_mid(s, dinv, b, W):
    return pl.pallas_call(
        _mid_body,
        grid=(GRID,),
        in_specs=[_SPLIT_SPEC, _row_spec(1), _FULL((1, DH)), _FULL((DH, DH))],
        out_specs=_SPLIT_SPEC,
        out_shape=jax.ShapeDtypeStruct((2, NP, HALF), jnp.float32),
    )(s, dinv, b, W)


def _tc_pool(s, dinv, b, batch_p, fw1, fb1, fw2, fb2):
    return pl.pallas_call(
        _pool_body,
        grid=(GRID,),
        in_specs=[_SPLIT_SPEC, _row_spec(1), _FULL((1, DH)),
                  _row_spec(1), _FULL((DH, DH)), _FULL((1, DH)),
                  _FULL((DH, D_OUT)), _FULL((1, D_OUT))],
        out_specs=pl.BlockSpec((NG, D_OUT), lambda i: (0, 0)),
        out_shape=jax.ShapeDtypeStruct((NG, D_OUT), jnp.float32),
        scratch_shapes=[pltpu.VMEM((NG, DH), jnp.float32),
                        pltpu.VMEM((NG, 1), jnp.float32)],
    )(s, dinv, b, batch_p, fw1, fb1, fw2, fb2)


def kernel(x, edge_index, batch, W1, b1, W2, b2, W3, b3, W4, b4, W5, b5,
           fw1, fb1, fw2, fb2):
    src = edge_index[0]
    dst = edge_index[1]
    xp = jnp.pad(x, ((0, NP - N), (0, 0)))
    batch_p = jnp.pad(batch, (0, NP - N), constant_values=NG).reshape(NP, 1)

    deg0, deg1 = _sc_deg()(dst)
    g, dinv = _tc_first(xp, W1, deg0.reshape(NP, 1), deg1.reshape(NP, 1))
    for Wn, bp in ((W2, b1), (W3, b2), (W4, b3), (W5, b4)):
        s = _sc_agg()(g, src, dst).reshape(2, NP, HALF)
        g = _tc_mid(s, dinv, bp.reshape(1, DH), Wn)
    s = _sc_agg()(g, src, dst).reshape(2, NP, HALF)
    return _tc_pool(s, dinv, b5.reshape(1, DH), batch_p, fw1,
                    fb1.reshape(1, DH), fw2, fb2.reshape(1, D_OUT))
